# baseline (speedup 1.0000x reference)
import functools

import jax
import jax.numpy as jnp
from jax import lax
from jax.experimental import pallas as pl


def _body(n, cu_ref, qsl_ref, out_ref):
    cu = cu_ref[...][None, :]
    w = qsl_ref[...][None, :] - cu
    i_col = lax.broadcasted_iota(jnp.int32, (n, n + 1), 0)
    j_row = lax.broadcasted_iota(jnp.int32, (n, n + 1), 1)
    b = jnp.sum(jnp.where(cu <= i_col, 1, 0), axis=1, keepdims=True)
    b = jnp.minimum(b, n)
    s = jnp.sum(jnp.where(j_row == b, w, 0), axis=1, keepdims=True)
    out_ref[...] = i_col[:, :1] + s


@functools.partial(jax.jit, static_argnums=(2,))
def _logits_indices_tc(cu, qsl, n):
    out = pl.pallas_call(
        functools.partial(_body, n),
        out_shape=jax.ShapeDtypeStruct((n, 1), jnp.int32),
    )(cu, qsl)
    return out.reshape(n)


def kernel(input_ids, idx_mapping, last_sampled_tokens, query_start_loc,
           seq_lens, prefill_len, draft_tokens, cu_num_logits, num_logits):
    n = cu_num_logits.shape[0] - 1
    return _logits_indices_tc(cu_num_logits.astype(jnp.int32),
                              query_start_loc.astype(jnp.int32), n)


# speedup vs baseline: 2.1789x; 2.1789x over previous
import functools

import jax
import jax.numpy as jnp
from jax import lax
from jax.experimental import pallas as pl


def _body(n, cu_ref, qsl_ref, out_ref):
    m = n + 1
    cu = cu_ref[...][None, :]
    w = qsl_ref[...][None, :] - cu
    jr = lax.broadcasted_iota(jnp.int32, (m, m), 0)
    jc = lax.broadcasted_iota(jnp.int32, (m, m), 1)
    eye = jr == jc
    cu_col = jnp.sum(jnp.where(eye, cu, 0), axis=1, keepdims=True)
    w_col = jnp.sum(jnp.where(eye, w, 0), axis=1, keepdims=True)
    ii = lax.broadcasted_iota(jnp.int32, (m, n), 1)
    jj = lax.broadcasted_iota(jnp.int32, (m, n), 0)
    b = jnp.sum(jnp.where(cu_col <= ii, 1, 0), axis=0, keepdims=True)
    b = jnp.minimum(b, n)
    s = jnp.sum(jnp.where(jj == b, w_col, 0), axis=0, keepdims=True)
    out_ref[...] = ii[:1, :] + s


@functools.partial(jax.jit, static_argnums=(2,))
def _logits_indices_tc(cu, qsl, n):
    out = pl.pallas_call(
        functools.partial(_body, n),
        out_shape=jax.ShapeDtypeStruct((1, n), jnp.int32),
    )(cu, qsl)
    return out.reshape(n)


def kernel(input_ids, idx_mapping, last_sampled_tokens, query_start_loc,
           seq_lens, prefill_len, draft_tokens, cu_num_logits, num_logits):
    n = cu_num_logits.shape[0] - 1
    return _logits_indices_tc(cu_num_logits.astype(jnp.int32),
                              query_start_loc.astype(jnp.int32), n)
